# Initial kernel scaffold; baseline (speedup 1.0000x reference)
import jax
import jax.numpy as jnp
from jax.experimental import pallas as pl


def kernel(x, edge_index, edge_attr, batch, W1_0, b1_0, g_0, beta_0, W2_0, b2_0, W1_1, b1_1, g_1, beta_1, W2_1, b2_1, W1_2, b1_2, g_2, beta_2, W2_2, b2_2, We, bWe):
    raise NotImplementedError("write your pallas kernel here")


# speedup vs baseline: 3.5859x; 3.5859x over previous
import jax
import jax.numpy as jnp
from jax import lax
from jax.experimental import pallas as pl
from jax.experimental.pallas import tpu as pltpu
from jax.experimental.pallas import tpu_sc as plsc

N = 10000
E = 320000
D = 128
ED = 16
B = 64

NC, NS = 2, 16
NW = NC * NS
EPW = E // NW
C = 80
NCHUNK = EPW // C
NPAD = 10240
RPT = NPAD // NS


def _make_segsum(with_edge):
    mesh = plsc.VectorSubcoreMesh(core_axis_name="c", subcore_axis_name="s")
    scratch = [
        pltpu.VMEM((C,), jnp.int32),
        pltpu.VMEM((C,), jnp.int32),
        pltpu.VMEM((C, D), jnp.float32),
    ]
    if with_edge:
        scratch.append(pltpu.VMEM((C, D), jnp.float32))
    scratch += [
        pltpu.VMEM_SHARED((NPAD, D), jnp.float32),
        pltpu.SemaphoreType.DMA,
    ]

    def body(table, srcs, dsts, *rest):
        if with_edge:
            e_hbm, zeros, out, idx_s, idx_d, rows, ebuf, accum, sem = rest
        else:
            zeros, out, idx_s, idx_d, rows, accum, sem = rest
        c = lax.axis_index("c")
        s = lax.axis_index("s")
        w = s * NC + c
        pltpu.sync_copy(zeros.at[pl.ds(s * RPT, RPT)],
                        accum.at[pl.ds(s * RPT, RPT)])
        plsc.subcore_barrier()

        @pl.loop(0, NCHUNK)
        def _(i):
            base = w * EPW + i * C
            pltpu.sync_copy(srcs.at[pl.ds(base, C)], idx_s)
            pltpu.sync_copy(dsts.at[pl.ds(base, C)], idx_d)
            pltpu.async_copy(table.at[idx_s], rows, sem).wait()
            if with_edge:
                pltpu.sync_copy(e_hbm.at[pl.ds(base, C)], ebuf)

                @pl.loop(0, C)
                def _(r):
                    for j in range(D // 16):
                        sl = pl.ds(j * 16, 16)
                        v = rows[r, sl] + ebuf[r, sl]
                        rows[r, sl] = jnp.maximum(v, 0.0)
            pltpu.sync_copy(rows, accum.at[idx_d], add=True)

        plsc.subcore_barrier()
        pltpu.sync_copy(accum.at[pl.ds(s * RPT, RPT)],
                        out.at[c, pl.ds(s * RPT, RPT)])

    return pl.kernel(
        body,
        out_type=jax.ShapeDtypeStruct((NC, NPAD, D), jnp.float32),
        mesh=mesh,
        scratch_types=scratch,
    )


_segsum_plain = _make_segsum(False)
_segsum_edge = _make_segsum(True)


_BLK = 400


def _mlp(h, a, W1, b1, g, beta, W2, b2, post_relu):
    def body(h_ref, a_ref, W1_ref, b1_ref, g_ref, beta_ref, W2_ref, b2_ref,
             o_ref):
        t = h_ref[...] + a_ref[0] + a_ref[1]
        u = jnp.dot(t, W1_ref[...], preferred_element_type=jnp.float32)
        u = g_ref[...] * (u + b1_ref[...]) + beta_ref[...]
        u = jnp.maximum(u, 0.0)
        o = jnp.dot(u, W2_ref[...], preferred_element_type=jnp.float32)
        o = o + b2_ref[...]
        if post_relu:
            o = jnp.maximum(o, 0.0)
        o_ref[...] = o

    return pl.pallas_call(
        body,
        grid=(N // _BLK,),
        in_specs=[
            pl.BlockSpec((_BLK, D), lambda i: (i, 0)),
            pl.BlockSpec((NC, _BLK, D), lambda i: (0, i, 0)),
            pl.BlockSpec((D, D), lambda i: (0, 0)),
            pl.BlockSpec((1, D), lambda i: (0, 0)),
            pl.BlockSpec((1, D), lambda i: (0, 0)),
            pl.BlockSpec((1, D), lambda i: (0, 0)),
            pl.BlockSpec((D, D), lambda i: (0, 0)),
            pl.BlockSpec((1, D), lambda i: (0, 0)),
        ],
        out_specs=pl.BlockSpec((_BLK, D), lambda i: (i, 0)),
        out_shape=jax.ShapeDtypeStruct((N, D), jnp.float32),
    )(h, a, W1, b1.reshape(1, D), g.reshape(1, D), beta.reshape(1, D), W2,
      b2.reshape(1, D))


_EBLK = 4000


def _edge_lin(edge_attr, We, bWe):
    def body(ea_ref, We_ref, b_ref, o_ref):
        o_ref[...] = (jnp.dot(ea_ref[...], We_ref[...],
                              preferred_element_type=jnp.float32)
                      + b_ref[...])

    return pl.pallas_call(
        body,
        grid=(E // _EBLK,),
        in_specs=[
            pl.BlockSpec((_EBLK, ED), lambda i: (i, 0)),
            pl.BlockSpec((ED, D), lambda i: (0, 0)),
            pl.BlockSpec((1, D), lambda i: (0, 0)),
        ],
        out_specs=pl.BlockSpec((_EBLK, D), lambda i: (i, 0)),
        out_shape=jax.ShapeDtypeStruct((E, D), jnp.float32),
    )(edge_attr, We, bWe.reshape(1, D))


def _pool(h, batch):
    nblk = N // _BLK

    def body(h_ref, b_ref, o_ref, cnt_ref):
        i = pl.program_id(0)

        @pl.when(i == 0)
        def _():
            o_ref[...] = jnp.zeros_like(o_ref)
            cnt_ref[...] = jnp.zeros_like(cnt_ref)

        bb = b_ref[0, 0, :]
        iota = lax.broadcasted_iota(jnp.int32, (_BLK, B), 1)
        onehot = (bb[:, None] == iota).astype(jnp.float32)
        dn = (((0,), (0,)), ((), ()))
        o_ref[...] += lax.dot_general(onehot, h_ref[...], dn,
                                      preferred_element_type=jnp.float32)
        cnt_ref[...] += lax.dot_general(
            onehot, jnp.ones((_BLK, D), jnp.float32), dn,
            preferred_element_type=jnp.float32)

        @pl.when(i == nblk - 1)
        def _():
            o_ref[...] = o_ref[...] / jnp.maximum(cnt_ref[...], 1.0)

    return pl.pallas_call(
        body,
        grid=(nblk,),
        in_specs=[
            pl.BlockSpec((_BLK, D), lambda i: (i, 0)),
            pl.BlockSpec((1, 1, _BLK), lambda i: (i, 0, 0)),
        ],
        out_specs=pl.BlockSpec((B, D), lambda i: (0, 0)),
        out_shape=jax.ShapeDtypeStruct((B, D), jnp.float32),
        scratch_shapes=[pltpu.VMEM((B, D), jnp.float32)],
    )(h, batch.reshape(N // _BLK, 1, _BLK))


def kernel(x, edge_index, edge_attr, batch, W1_0, b1_0, g_0, beta_0, W2_0,
           b2_0, W1_1, b1_1, g_1, beta_1, W2_1, b2_1, W1_2, b1_2, g_2, beta_2,
           W2_2, b2_2, We, bWe):
    src = edge_index[0]
    dst = edge_index[1]
    zeros = jnp.zeros((NPAD, D), jnp.float32)

    a = _segsum_plain(x, src, dst, zeros)
    h = _mlp(x, a, W1_0, b1_0, g_0, beta_0, W2_0, b2_0, post_relu=True)

    e = _edge_lin(edge_attr, We, bWe)
    a = _segsum_edge(h, src, dst, e, zeros)
    h = _mlp(h, a, W1_1, b1_1, g_1, beta_1, W2_1, b2_1, post_relu=True)

    a = _segsum_plain(h, src, dst, zeros)
    h = _mlp(h, a, W1_2, b1_2, g_2, beta_2, W2_2, b2_2, post_relu=False)

    return _pool(h, batch)


# speedup vs baseline: 4.1614x; 1.1605x over previous
import jax
import jax.numpy as jnp
from jax import lax
from jax.experimental import pallas as pl
from jax.experimental.pallas import tpu as pltpu
from jax.experimental.pallas import tpu_sc as plsc

N = 10000
E = 320000
D = 128
ED = 16
B = 64

NC, NS = 2, 16
NW = NC * NS
EPW = E // NW
C = 40
NCHUNK = EPW // C
NPAD = 10240
RPT = NPAD // NS


NB = 2
NROUND = NCHUNK // NB


def _make_segsum(with_edge):
    mesh = plsc.VectorSubcoreMesh(core_axis_name="c", subcore_axis_name="s")
    scratch = [
        pltpu.VMEM((2, NB, C), jnp.int32),
        pltpu.VMEM((2, NB, C), jnp.int32),
        [pltpu.VMEM((C, D), jnp.float32) for _ in range(NB)],
    ]
    if with_edge:
        scratch.append([pltpu.VMEM((C, D), jnp.float32) for _ in range(NB)])
    scratch += [
        pltpu.VMEM_SHARED((NPAD, D), jnp.float32),
        pltpu.SemaphoreType.DMA((NB,)),
        pltpu.SemaphoreType.DMA((NB,)),
        pltpu.SemaphoreType.DMA,
    ]
    if with_edge:
        scratch.append(pltpu.SemaphoreType.DMA((NB,)))

    def body(table, srcs3, dsts3, *rest):
        if with_edge:
            (e_hbm, zeros, out, idx_s, idx_d, rows, ebuf, accum, gsem, ssem,
             isem, esem) = rest
        else:
            zeros, out, idx_s, idx_d, rows, accum, gsem, ssem, isem = rest
        c = lax.axis_index("c")
        s = lax.axis_index("s")
        w = s * NC + c
        pltpu.sync_copy(zeros.at[pl.ds(s * RPT, RPT)],
                        accum.at[pl.ds(s * RPT, RPT)])
        pltpu.sync_copy(srcs3.at[w, 0], idx_s.at[0])
        pltpu.sync_copy(dsts3.at[w, 0], idx_d.at[0])
        plsc.subcore_barrier()

        def gstart(chunk, slot, b):
            pltpu.async_copy(table.at[idx_s.at[slot, b]], rows[b], gsem.at[b])
            if with_edge:
                pltpu.async_copy(e_hbm.at[pl.ds(w * EPW + chunk * C, C)],
                                 ebuf[b], esem.at[b])

        def gwait(b):
            pltpu.make_async_copy(table.at[idx_s.at[0, 0]], rows[b],
                                  gsem.at[b]).wait()
            if with_edge:
                pltpu.make_async_copy(e_hbm.at[pl.ds(0, C)], ebuf[b],
                                      esem.at[b]).wait()

        def compute(b):
            if with_edge:
                @pl.loop(0, C, unroll=4)
                def _(r):
                    for j in range(D // 16):
                        sl = pl.ds(j * 16, 16)
                        rows[b][r, sl] = jnp.maximum(
                            rows[b][r, sl] + ebuf[b][r, sl], 0.0)

        def sstart(slot, b):
            pltpu.async_copy(rows[b], accum.at[idx_d.at[slot, b]], ssem.at[b],
                             add=True)

        def swait(b):
            pltpu.make_async_copy(rows[b], accum.at[idx_d.at[0, 0]],
                                  ssem.at[b]).wait()

        for b in range(NB):
            gstart(b, 0, b)

        @pl.loop(0, NROUND - 1)
        def _(k):
            q = k % 2
            pltpu.async_copy(srcs3.at[w, k + 1], idx_s.at[1 - q], isem)
            pltpu.async_copy(dsts3.at[w, k + 1], idx_d.at[1 - q], isem)
            for b in range(NB):
                gwait(b)
                compute(b)
                sstart(q, b)
            pltpu.make_async_copy(srcs3.at[w, 0], idx_s.at[0], isem).wait()
            pltpu.make_async_copy(dsts3.at[w, 0], idx_d.at[0], isem).wait()
            for b in range(NB):
                swait(b)
                gstart((k + 1) * NB + b, 1 - q, b)

        ql = (NROUND - 1) % 2
        for b in range(NB):
            gwait(b)
            compute(b)
            sstart(ql, b)
        for b in range(NB):
            swait(b)

        plsc.subcore_barrier()
        pltpu.sync_copy(accum.at[pl.ds(s * RPT, RPT)],
                        out.at[c, pl.ds(s * RPT, RPT)])

    return pl.kernel(
        body,
        out_type=jax.ShapeDtypeStruct((NC, NPAD, D), jnp.float32),
        mesh=mesh,
        scratch_types=scratch,
    )


_segsum_plain = _make_segsum(False)
_segsum_edge = _make_segsum(True)


_BLK = 400


def _mlp(h, a, W1, b1, g, beta, W2, b2, post_relu):
    def body(h_ref, a_ref, W1_ref, b1_ref, g_ref, beta_ref, W2_ref, b2_ref,
             o_ref):
        t = h_ref[...] + a_ref[0] + a_ref[1]
        u = jnp.dot(t, W1_ref[...], preferred_element_type=jnp.float32)
        u = g_ref[...] * (u + b1_ref[...]) + beta_ref[...]
        u = jnp.maximum(u, 0.0)
        o = jnp.dot(u, W2_ref[...], preferred_element_type=jnp.float32)
        o = o + b2_ref[...]
        if post_relu:
            o = jnp.maximum(o, 0.0)
        o_ref[...] = o

    return pl.pallas_call(
        body,
        grid=(N // _BLK,),
        in_specs=[
            pl.BlockSpec((_BLK, D), lambda i: (i, 0)),
            pl.BlockSpec((NC, _BLK, D), lambda i: (0, i, 0)),
            pl.BlockSpec((D, D), lambda i: (0, 0)),
            pl.BlockSpec((1, D), lambda i: (0, 0)),
            pl.BlockSpec((1, D), lambda i: (0, 0)),
            pl.BlockSpec((1, D), lambda i: (0, 0)),
            pl.BlockSpec((D, D), lambda i: (0, 0)),
            pl.BlockSpec((1, D), lambda i: (0, 0)),
        ],
        out_specs=pl.BlockSpec((_BLK, D), lambda i: (i, 0)),
        out_shape=jax.ShapeDtypeStruct((N, D), jnp.float32),
    )(h, a, W1, b1.reshape(1, D), g.reshape(1, D), beta.reshape(1, D), W2,
      b2.reshape(1, D))


_EBLK = 4000


def _edge_lin(edge_attr, We, bWe):
    def body(ea_ref, We_ref, b_ref, o_ref):
        o_ref[...] = (jnp.dot(ea_ref[...], We_ref[...],
                              preferred_element_type=jnp.float32)
                      + b_ref[...])

    return pl.pallas_call(
        body,
        grid=(E // _EBLK,),
        in_specs=[
            pl.BlockSpec((_EBLK, ED), lambda i: (i, 0)),
            pl.BlockSpec((ED, D), lambda i: (0, 0)),
            pl.BlockSpec((1, D), lambda i: (0, 0)),
        ],
        out_specs=pl.BlockSpec((_EBLK, D), lambda i: (i, 0)),
        out_shape=jax.ShapeDtypeStruct((E, D), jnp.float32),
    )(edge_attr, We, bWe.reshape(1, D))


def _pool(h, batch):
    nblk = N // _BLK

    def body(h_ref, b_ref, o_ref, cnt_ref):
        i = pl.program_id(0)

        @pl.when(i == 0)
        def _():
            o_ref[...] = jnp.zeros_like(o_ref)
            cnt_ref[...] = jnp.zeros_like(cnt_ref)

        bb = b_ref[0, 0, :]
        iota = lax.broadcasted_iota(jnp.int32, (_BLK, B), 1)
        onehot = (bb[:, None] == iota).astype(jnp.float32)
        dn = (((0,), (0,)), ((), ()))
        o_ref[...] += lax.dot_general(onehot, h_ref[...], dn,
                                      preferred_element_type=jnp.float32)
        cnt_ref[...] += lax.dot_general(
            onehot, jnp.ones((_BLK, D), jnp.float32), dn,
            preferred_element_type=jnp.float32)

        @pl.when(i == nblk - 1)
        def _():
            o_ref[...] = o_ref[...] / jnp.maximum(cnt_ref[...], 1.0)

    return pl.pallas_call(
        body,
        grid=(nblk,),
        in_specs=[
            pl.BlockSpec((_BLK, D), lambda i: (i, 0)),
            pl.BlockSpec((1, 1, _BLK), lambda i: (i, 0, 0)),
        ],
        out_specs=pl.BlockSpec((B, D), lambda i: (0, 0)),
        out_shape=jax.ShapeDtypeStruct((B, D), jnp.float32),
        scratch_shapes=[pltpu.VMEM((B, D), jnp.float32)],
    )(h, batch.reshape(N // _BLK, 1, _BLK))


def kernel(x, edge_index, edge_attr, batch, W1_0, b1_0, g_0, beta_0, W2_0,
           b2_0, W1_1, b1_1, g_1, beta_1, W2_1, b2_1, W1_2, b1_2, g_2, beta_2,
           W2_2, b2_2, We, bWe):
    src = edge_index[0].reshape(NW, NROUND, NB, C)
    dst = edge_index[1].reshape(NW, NROUND, NB, C)
    zeros = jnp.zeros((NPAD, D), jnp.float32)

    a = _segsum_plain(x, src, dst, zeros)
    h = _mlp(x, a, W1_0, b1_0, g_0, beta_0, W2_0, b2_0, post_relu=True)

    e = _edge_lin(edge_attr, We, bWe)
    a = _segsum_edge(h, src, dst, e, zeros)
    h = _mlp(h, a, W1_1, b1_1, g_1, beta_1, W2_1, b2_1, post_relu=True)

    a = _segsum_plain(h, src, dst, zeros)
    h = _mlp(h, a, W1_2, b1_2, g_2, beta_2, W2_2, b2_2, post_relu=False)

    return _pool(h, batch)
